# baseline (speedup 1.0000x reference)
import functools

import jax
import jax.numpy as jnp
from jax import lax
from jax.experimental import pallas as pl
from jax.experimental.pallas import tpu as pltpu
from jax.experimental.pallas import tpu_sc as plsc

N_LAYER = 24
N_HEAD = 16
HEAD_SIZE = 64
TOTAL_DIM = 3072
OUT_DIM = TOTAL_DIM // 3
T = 4096
T_CHUNKS = 8

SC_LAYERS = 4
TC_LAYERS = N_LAYER - SC_LAYERS
SC_ROWS = SC_LAYERS * OUT_DIM
SC_BASE = TC_LAYERS * OUT_DIM

NC = 2
NS = 16
NW = NC * NS
L = 16

RS = SC_ROWS // NW
CR = 16
N_CHUNK = RS // CR
RB = 8
NV = TOTAL_DIM // L


def _mean_body(e_ref, out_ref):
    t = pl.program_id(0)

    @pl.when(t == 0)
    def _init():
        out_ref[...] = jnp.zeros_like(out_ref)

    out_ref[...] += jnp.sum(e_ref[...], axis=0, keepdims=True) * (1.0 / T)


def _tc_mv_body(vec_ref, w_ref, b_ref, out_ref):
    out_ref[0] = (
        jax.lax.dot_general(
            vec_ref[...], w_ref[0], (((1,), (1,)), ((), ())),
            preferred_element_type=jnp.float32,
        )
        + b_ref[0]
    )


def _hsum_bcast(v):
    lane = lax.iota(jnp.int32, L)
    for sh in (1, 2, 4, 8):
        idx = lane ^ sh
        v = v + v.at[idx].get(mode="promise_in_bounds", unique_indices=True)
    return v


def _sc_mv_body(w_hbm, vec_hbm, b_hbm, out_hbm, vec_v, wbuf0, wbuf1,
                bbuf, outb, sem0, sem1):
    wid = lax.axis_index("s") * NC + lax.axis_index("c")
    rowbase = pl.multiple_of(SC_BASE + wid * RS, CR)

    pltpu.sync_copy(vec_hbm, vec_v)
    pltpu.sync_copy(b_hbm.at[pl.ds(rowbase, RS)], bbuf)

    bufs = (wbuf0, wbuf1)
    sems = (sem0, sem1)

    def issue(c):
        return pltpu.async_copy(
            w_hbm.at[pl.ds(rowbase + c * CR, CR)], bufs[c % 2], sems[c % 2]
        )

    handles = [None, None]
    handles[0] = issue(0)
    for c in range(N_CHUNK):
        if c + 1 < N_CHUNK:
            handles[(c + 1) % 2] = issue(c + 1)
        handles[c % 2].wait()
        buf = bufs[c % 2]
        row_sums = []
        for rb in range(CR // RB):
            r0 = rb * RB

            def col_body(j, accs, _buf=buf, _r0=r0):
                jj = pl.multiple_of(j * L, L)
                vv = vec_v[pl.ds(jj, L)]
                return tuple(
                    accs[r] + _buf[_r0 + r, pl.ds(jj, L)] * vv
                    for r in range(RB)
                )

            accs = lax.fori_loop(
                0, NV, col_body,
                tuple(jnp.zeros((L,), jnp.float32) for _ in range(RB)),
            )
            row_sums.extend(_hsum_bcast(a) for a in accs)
        lane = lax.iota(jnp.int32, L)
        resv = jnp.zeros((L,), jnp.float32)
        for r, tv in enumerate(row_sums):
            resv = jnp.where(lane == r, tv, resv)
        outb[pl.ds(c * CR, CR)] = resv

    for q in range(RS // L):
        outb[pl.ds(q * L, L)] = outb[pl.ds(q * L, L)] + bbuf[pl.ds(q * L, L)]
    pltpu.sync_copy(outb, out_hbm.at[pl.ds(pl.multiple_of(wid * RS, L), RS)])


_sc_matvec = functools.partial(
    pl.kernel,
    out_type=jax.ShapeDtypeStruct((SC_ROWS,), jnp.float32),
    mesh=plsc.VectorSubcoreMesh(
        core_axis_name="c", subcore_axis_name="s",
        num_cores=NC, num_subcores=NS,
    ),
    scratch_types=[
        pltpu.VMEM((TOTAL_DIM,), jnp.float32),
        pltpu.VMEM((CR, TOTAL_DIM), jnp.float32),
        pltpu.VMEM((CR, TOTAL_DIM), jnp.float32),
        pltpu.VMEM((RS,), jnp.float32),
        pltpu.VMEM((RS,), jnp.float32),
        pltpu.SemaphoreType.DMA,
        pltpu.SemaphoreType.DMA,
    ],
)(_sc_mv_body)


def kernel(system_emb, W_proj, b_proj):
    e = system_emb.reshape(T, TOTAL_DIM)
    vec = pl.pallas_call(
        _mean_body,
        grid=(T_CHUNKS,),
        in_specs=[pl.BlockSpec((T // T_CHUNKS, TOTAL_DIM), lambda t: (t, 0))],
        out_specs=pl.BlockSpec((1, TOTAL_DIM), lambda t: (0, 0)),
        out_shape=jax.ShapeDtypeStruct((1, TOTAL_DIM), jnp.float32),
    )(e)

    w_flat = W_proj.reshape(N_LAYER * OUT_DIM, TOTAL_DIM)
    b_flat = b_proj.reshape(N_LAYER * OUT_DIM)
    out_sc = _sc_matvec(w_flat, vec.reshape(TOTAL_DIM), b_flat)

    out_tc = pl.pallas_call(
        _tc_mv_body,
        grid=(TC_LAYERS,),
        in_specs=[
            pl.BlockSpec((1, TOTAL_DIM), lambda l: (0, 0)),
            pl.BlockSpec((1, OUT_DIM, TOTAL_DIM), lambda l: (l, 0, 0)),
            pl.BlockSpec((1, 1, OUT_DIM), lambda l: (l, 0, 0)),
        ],
        out_specs=pl.BlockSpec((1, 1, OUT_DIM), lambda l: (l, 0, 0)),
        out_shape=jax.ShapeDtypeStruct((TC_LAYERS, 1, OUT_DIM), jnp.float32),
    )(vec, W_proj, b_proj.reshape(N_LAYER, 1, OUT_DIM))

    out = jnp.concatenate(
        [out_tc.reshape(TC_LAYERS, OUT_DIM), out_sc.reshape(SC_LAYERS, OUT_DIM)],
        axis=0,
    )
    return out.reshape(N_LAYER, N_HEAD, HEAD_SIZE)


# speedup vs baseline: 1.1265x; 1.1265x over previous
import jax
import jax.numpy as jnp
from jax.experimental import pallas as pl
from jax.experimental.pallas import tpu as pltpu

N_LAYER = 24
N_HEAD = 16
HEAD_SIZE = 64
TOTAL_DIM = 3072
OUT_DIM = TOTAL_DIM // 3
T = 4096
T_CHUNKS = 16
HALF = N_LAYER // 2


def _body(e_ref, w1_ref, w2_ref, b1_ref, b2_ref, o1_ref, o2_ref, vec_ref):
    t = pl.program_id(0)

    @pl.when(t == 0)
    def _init():
        vec_ref[...] = jnp.zeros_like(vec_ref)

    @pl.when(t < T_CHUNKS)
    def _mean():
        vec_ref[...] += jnp.sum(e_ref[...], axis=0, keepdims=True) * (1.0 / T)

    @pl.when(t >= T_CHUNKS)
    def _matvec():
        v = vec_ref[...]
        o1_ref[0] = (
            jax.lax.dot_general(v, w1_ref[0], (((1,), (1,)), ((), ())),
                                preferred_element_type=jnp.float32)
            + b1_ref[0]
        )
        o2_ref[0] = (
            jax.lax.dot_general(v, w2_ref[0], (((1,), (1,)), ((), ())),
                                preferred_element_type=jnp.float32)
            + b2_ref[0]
        )


def kernel(system_emb, W_proj, b_proj):
    e = system_emb.reshape(T, TOTAL_DIM)
    b3 = b_proj.reshape(N_LAYER, 1, OUT_DIM)
    out1, out2 = pl.pallas_call(
        _body,
        grid=(T_CHUNKS + HALF,),
        in_specs=[
            pl.BlockSpec((T // T_CHUNKS, TOTAL_DIM),
                         lambda t: (jnp.minimum(t, T_CHUNKS - 1), 0)),
            pl.BlockSpec((1, OUT_DIM, TOTAL_DIM),
                         lambda t: (jnp.maximum(t - T_CHUNKS, 0), 0, 0)),
            pl.BlockSpec((1, OUT_DIM, TOTAL_DIM),
                         lambda t: (jnp.maximum(t - T_CHUNKS, 0) + HALF, 0, 0)),
            pl.BlockSpec((1, 1, OUT_DIM),
                         lambda t: (jnp.maximum(t - T_CHUNKS, 0), 0, 0)),
            pl.BlockSpec((1, 1, OUT_DIM),
                         lambda t: (jnp.maximum(t - T_CHUNKS, 0) + HALF, 0, 0)),
        ],
        out_specs=[
            pl.BlockSpec((1, 1, OUT_DIM),
                         lambda t: (jnp.maximum(t - T_CHUNKS, 0), 0, 0)),
            pl.BlockSpec((1, 1, OUT_DIM),
                         lambda t: (jnp.maximum(t - T_CHUNKS, 0), 0, 0)),
        ],
        out_shape=[
            jax.ShapeDtypeStruct((HALF, 1, OUT_DIM), jnp.float32),
            jax.ShapeDtypeStruct((HALF, 1, OUT_DIM), jnp.float32),
        ],
        scratch_shapes=[pltpu.VMEM((1, TOTAL_DIM), jnp.float32)],
    )(e, W_proj, W_proj, b3, b3)
    out = jnp.concatenate(
        [out1.reshape(HALF, OUT_DIM), out2.reshape(HALF, OUT_DIM)], axis=0
    )
    return out.reshape(N_LAYER, N_HEAD, HEAD_SIZE)


# speedup vs baseline: 1.1800x; 1.0475x over previous
import jax
import jax.numpy as jnp
from jax.experimental import pallas as pl
from jax.experimental.pallas import tpu as pltpu

N_LAYER = 24
N_HEAD = 16
HEAD_SIZE = 64
TOTAL_DIM = 3072
OUT_DIM = TOTAL_DIM // 3
T = 4096
T_CHUNKS = 8
HALF = N_LAYER // 2


def _body(e_ref, w1_ref, w2_ref, b1_ref, b2_ref, o1_ref, o2_ref, vec_ref):
    t = pl.program_id(0)

    @pl.when(t == 0)
    def _init():
        vec_ref[...] = jnp.zeros_like(vec_ref)

    @pl.when(t < T_CHUNKS)
    def _mean():
        vec_ref[...] += jnp.sum(e_ref[...], axis=0, keepdims=True) * (1.0 / T)

    @pl.when(t >= T_CHUNKS)
    def _matvec():
        v = vec_ref[...]
        q = jnp.maximum(t - T_CHUNKS, 0)
        r1 = jax.lax.dot_general(v, w1_ref[0], (((1,), (1,)), ((), ())),
                                 preferred_element_type=jnp.float32)
        r2 = jax.lax.dot_general(v, w2_ref[0], (((1,), (1,)), ((), ())),
                                 preferred_element_type=jnp.float32)

        @pl.when(q % 2 == 0)
        def _lo():
            o1_ref[0, 0, 0:OUT_DIM // 2] = r1[0] + b1_ref[0, 0, 0:OUT_DIM // 2]
            o2_ref[0, 0, 0:OUT_DIM // 2] = r2[0] + b2_ref[0, 0, 0:OUT_DIM // 2]

        @pl.when(q % 2 == 1)
        def _hi():
            o1_ref[0, 0, OUT_DIM // 2:] = r1[0] + b1_ref[0, 0, OUT_DIM // 2:]
            o2_ref[0, 0, OUT_DIM // 2:] = r2[0] + b2_ref[0, 0, OUT_DIM // 2:]


def kernel(system_emb, W_proj, b_proj):
    e = system_emb.reshape(T, TOTAL_DIM)
    b3 = b_proj.reshape(N_LAYER, 1, OUT_DIM)
    out1, out2 = pl.pallas_call(
        _body,
        grid=(T_CHUNKS + 2 * HALF,),
        in_specs=[
            pl.BlockSpec((T // T_CHUNKS, TOTAL_DIM),
                         lambda t: (jnp.minimum(t, T_CHUNKS - 1), 0)),
            pl.BlockSpec((1, OUT_DIM // 2, TOTAL_DIM),
                         lambda t: (jnp.maximum(t - T_CHUNKS, 0) // 2,
                                    jnp.maximum(t - T_CHUNKS, 0) % 2, 0)),
            pl.BlockSpec((1, OUT_DIM // 2, TOTAL_DIM),
                         lambda t: (jnp.maximum(t - T_CHUNKS, 0) // 2 + HALF,
                                    jnp.maximum(t - T_CHUNKS, 0) % 2, 0)),
            pl.BlockSpec((1, 1, OUT_DIM),
                         lambda t: (jnp.maximum(t - T_CHUNKS, 0) // 2, 0, 0)),
            pl.BlockSpec((1, 1, OUT_DIM),
                         lambda t: (jnp.maximum(t - T_CHUNKS, 0) // 2 + HALF, 0, 0)),
        ],
        out_specs=[
            pl.BlockSpec((1, 1, OUT_DIM),
                         lambda t: (jnp.maximum(t - T_CHUNKS, 0) // 2, 0, 0)),
            pl.BlockSpec((1, 1, OUT_DIM),
                         lambda t: (jnp.maximum(t - T_CHUNKS, 0) // 2, 0, 0)),
        ],
        out_shape=[
            jax.ShapeDtypeStruct((HALF, 1, OUT_DIM), jnp.float32),
            jax.ShapeDtypeStruct((HALF, 1, OUT_DIM), jnp.float32),
        ],
        scratch_shapes=[pltpu.VMEM((1, TOTAL_DIM), jnp.float32)],
    )(e, W_proj, W_proj, b3, b3)
    out = jnp.concatenate(
        [out1.reshape(HALF, OUT_DIM), out2.reshape(HALF, OUT_DIM)], axis=0
    )
    return out.reshape(N_LAYER, N_HEAD, HEAD_SIZE)
